# Initial kernel scaffold; baseline (speedup 1.0000x reference)
import jax
import jax.numpy as jnp
from jax.experimental import pallas as pl


def kernel(x, adjacency_matrix, W1, b1, W2, b2):
    raise NotImplementedError("write your pallas kernel here")


# speedup vs baseline: 5196.4915x; 5196.4915x over previous
import jax
import jax.numpy as jnp
from jax.experimental import pallas as pl


def _gcn2_kernel(x_ref, a_ref, w1_ref, b1_ref, w2_ref, b2_ref, out_ref):
    a = a_ref[...]
    n = a.shape[0]
    ones = jnp.ones((n, 1), dtype=a.dtype)
    deg = jax.lax.dot_general(
        a, ones, (((0,), (0,)), ((), ())),
        preferred_element_type=jnp.float32,
    ) + 1.0
    s = jnp.where(deg > 0, jax.lax.rsqrt(deg), 0.0)

    def layer(h_in, w_ref, b_ref):
        h = jnp.dot(h_in, w_ref[...], preferred_element_type=jnp.float32)
        hs = s * h
        m = jax.lax.dot_general(
            a, hs, (((0,), (0,)), ((), ())),
            preferred_element_type=jnp.float32,
        ) + hs
        return jax.nn.relu(s * m + b_ref[...])

    h1 = layer(x_ref[...], w1_ref, b1_ref)
    out_ref[...] = layer(h1, w2_ref, b2_ref)


def kernel(x, adjacency_matrix, W1, b1, W2, b2):
    n, d_out = x.shape[0], W2.shape[1]
    return pl.pallas_call(
        _gcn2_kernel,
        out_shape=jax.ShapeDtypeStruct((n, d_out), x.dtype),
    )(
        x,
        adjacency_matrix,
        W1,
        b1.reshape(1, -1),
        W2,
        b2.reshape(1, -1),
    )
